# baseline (speedup 1.0000x reference)
import functools

import numpy as np
import jax
import jax.numpy as jnp
from jax import lax
from jax.experimental import pallas as pl
from jax.experimental.pallas import tpu as pltpu
from jax.experimental.pallas import tpu_sc as plsc

N = 100000
D = 208
G = 512
NSCAL = 32
EPS = 1e-5

B = 128
NB = N // B
TAIL_BASE = N - B
TAIL_SKIP = NB * B - TAIL_BASE
NBT = NB + 1
NW = 32
MAXIT = -(-NBT // NW)
GP = 512
TW = 128
CA = 96
CB = D - CA
MAXJ = -(-NBT // 16)
TAIL_RG = TAIL_SKIP // 16
TFLAT = GP * TW
MFLAT = 32 * TW

_HI = jax.lax.Precision.HIGHEST


def _build_group_mat():
    gid = np.zeros((D,), np.int32)
    dcol = np.zeros((D,), np.float32)
    c = 0
    f = 0
    for mul, l in ((32, 0), (32, 1), (16, 2)):
        d = 2 * l + 1
        for _ in range(mul):
            for _ in range(d):
                gid[c] = f
                dcol[c] = d
                c += 1
            f += 1
    return (gid[:, None] == gid[None, :]).astype(np.float32) / dcol[None, :]


_M_CONST = _build_group_mat()

_MESH = plsc.VectorSubcoreMesh(core_axis_name="c", subcore_axis_name="s")


@functools.partial(
    pl.kernel,
    mesh=_MESH,
    out_type=jax.ShapeDtypeStruct((NW * TFLAT,), jnp.float32),
    scratch_types=[
        pltpu.VMEM((B * D,), jnp.float32),
        pltpu.VMEM((B,), jnp.int32),
        pltpu.VMEM((TFLAT,), jnp.float32),
    ],
    compiler_params=pltpu.CompilerParams(needs_layout_passes=False),
)
def _sc_stats(x_hbm, b_hbm, out_hbm, xb, idxb, table):
    cid = lax.axis_index("c")
    sid = lax.axis_index("s")
    half = sid % 2
    g8 = sid // 2
    zero16 = jnp.zeros((16,), jnp.float32)
    ones16 = jnp.full((16,), 1.0, jnp.float32)
    lane = lax.iota(jnp.int32, 16)

    def zrow(k, c_):
        table[pl.ds(k * 16, 16)] = zero16
        return c_

    lax.fori_loop(0, TFLAT // 16, zrow, 0)

    def block_body(j, carry):
        b = cid + 2 * g8 + 16 * j

        @pl.when(b < NBT)
        def _():
            base = jnp.where(b == NB, TAIL_BASE, b * B)
            pltpu.sync_copy(b_hbm.at[pl.ds(base, B)], idxb)
            pltpu.sync_copy(x_hbm.at[pl.ds(base * D, B * D)], xb)
            rg0 = jnp.where(b == NB, TAIL_RG, 0)

            @pl.when(half == 0)
            def _():
                def rg_body(rg, c_):
                    gv = idxb[pl.ds(rg * 16, 16)] * TW
                    rowv = (rg * 16 + lane) * D
                    for c in range(CA):
                        v = plsc.load_gather(xb, [rowv + c])
                        plsc.addupdate_scatter(table, [gv + c], v * v)
                        if c < NSCAL:
                            plsc.addupdate_scatter(table, [gv + (CA + c)], v)
                    return c_

                lax.fori_loop(rg0, B // 16, rg_body, 0)

            @pl.when(half == 1)
            def _():
                def rg_body(rg, c_):
                    gv = idxb[pl.ds(rg * 16, 16)] * TW
                    rowv = (rg * 16 + lane) * D
                    for c in range(CB):
                        v = plsc.load_gather(xb, [rowv + (CA + c)])
                        plsc.addupdate_scatter(table, [gv + c], v * v)
                    plsc.addupdate_scatter(table, [gv + CB], ones16)
                    return c_

                lax.fori_loop(rg0, B // 16, rg_body, 0)

        return carry

    lax.fori_loop(0, MAXJ, block_body, 0)

    pltpu.sync_copy(table, out_hbm.at[pl.ds((cid * 16 + sid) * TFLAT, TFLAT)])


def _coeff_kernel(s_ref, m_ref, w_ref, bb_ref, coef_ref):
    st = s_ref[...]
    a = jnp.zeros((GP, TW), jnp.float32)
    bt = jnp.zeros((GP, TW), jnp.float32)
    for t in range(NW):
        part = st[t * GP:(t + 1) * GP]
        if t % 2 == 0:
            a = a + part
        else:
            bt = bt + part
    cnt = jnp.maximum(bt[:, CB:CB + 1], 1.0)
    mean = a[:, CA:CA + NSCAL] / cnt
    sq = jnp.concatenate([a[:, :CA], bt[:, :CB]], axis=1)
    gs = jax.lax.dot_general(
        sq, m_ref[...], (((1,), (0,)), ((), ())),
        preferred_element_type=jnp.float32, precision=_HI) / cnt
    meanp = jnp.concatenate(
        [mean, jnp.zeros((GP, D - NSCAL), jnp.float32)], axis=1)
    var = gs - meanp * meanp
    scale = w_ref[...] * jax.lax.rsqrt(var + EPS)
    shift = bb_ref[...] - mean * scale[:, :NSCAL]
    coef_ref[...] = jnp.concatenate(
        [scale, shift, jnp.zeros((GP, 16), jnp.float32)], axis=1)


@functools.partial(
    pl.kernel,
    mesh=_MESH,
    out_type=jax.ShapeDtypeStruct((N * D,), jnp.float32),
    scratch_types=[
        pltpu.VMEM((B * D,), jnp.float32),
        pltpu.VMEM((B, 256), jnp.float32),
        pltpu.VMEM((B,), jnp.int32),
        pltpu.SemaphoreType.DMA,
    ],
    compiler_params=pltpu.CompilerParams(needs_layout_passes=False),
)
def _sc_apply(x_hbm, b_hbm, coef_hbm, out_hbm, xb, cb, idxb, sem):
    cid = lax.axis_index("c")
    sid = lax.axis_index("s")
    wid = sid * 2 + cid

    def block_body(it, carry):
        b = wid + it * NW

        @pl.when(b < NBT)
        def _():
            base = jnp.where(b == NB, TAIL_BASE, b * B)
            pltpu.sync_copy(x_hbm.at[pl.ds(base * D, B * D)], xb)
            pltpu.sync_copy(b_hbm.at[pl.ds(base, B)], idxb)
            pltpu.async_copy(coef_hbm.at[idxb], cb, sem).wait()

            def row_body(r, rc):
                for c in range(D // 16):
                    v = xb[pl.ds(r * D + c * 16, 16)] * cb[r, pl.ds(c * 16, 16)]
                    if c < NSCAL // 16:
                        v = v + cb[r, pl.ds(D + c * 16, 16)]
                    xb[pl.ds(r * D + c * 16, 16)] = v
                return rc

            lax.fori_loop(0, B, row_body, 0)
            pltpu.sync_copy(xb, out_hbm.at[pl.ds(base * D, B * D)])

        return carry

    lax.fori_loop(0, MAXIT, block_body, 0)


@jax.jit
def kernel(input, batch, weight, bias):
    bi = batch.astype(jnp.int32)
    xf = input.reshape(N * D)
    wcol = jnp.concatenate(
        [weight[:32], jnp.repeat(weight[32:64], 3), jnp.repeat(weight[64:], 5)]
    ).reshape(1, D)
    bb = bias.reshape(1, NSCAL)
    m = jnp.asarray(_M_CONST)

    stats = _sc_stats(xf, bi)

    coef = pl.pallas_call(
        _coeff_kernel,
        out_shape=jax.ShapeDtypeStruct((GP, 256), jnp.float32),
    )(stats.reshape(NW * GP, TW), m, wcol, bb)

    return _sc_apply(xf, bi, coef).reshape(N, D)


# speedup vs baseline: 2.5785x; 2.5785x over previous
import functools

import numpy as np
import jax
import jax.numpy as jnp
from jax import lax
from jax.experimental import pallas as pl
from jax.experimental.pallas import tpu as pltpu
from jax.experimental.pallas import tpu_sc as plsc

N = 100000
D = 208
G = 512
NSCAL = 32
EPS = 1e-5

B = 128
NB = N // B
TAIL_BASE = N - B
TAIL_SKIP = NB * B - TAIL_BASE
NBT = NB + 1
NW = 32
MAXIT = -(-NBT // NW)
GP = 512
TW = 128
CA = 96
CB = D - CA
MAXJ = -(-NBT // 16)
TAIL_RG = TAIL_SKIP // 16
TFLAT = GP * TW
MFLAT = 32 * TW

_HI = jax.lax.Precision.HIGHEST


def _build_group_mat():
    gid = np.zeros((D,), np.int32)
    dcol = np.zeros((D,), np.float32)
    c = 0
    f = 0
    for mul, l in ((32, 0), (32, 1), (16, 2)):
        d = 2 * l + 1
        for _ in range(mul):
            for _ in range(d):
                gid[c] = f
                dcol[c] = d
                c += 1
            f += 1
    return (gid[:, None] == gid[None, :]).astype(np.float32) / dcol[None, :]


_M_CONST = _build_group_mat()

_MESH = plsc.VectorSubcoreMesh(core_axis_name="c", subcore_axis_name="s")


@functools.partial(
    pl.kernel,
    mesh=_MESH,
    out_type=jax.ShapeDtypeStruct((NW * TFLAT,), jnp.float32),
    scratch_types=[
        pltpu.VMEM((B, D), jnp.float32),
        pltpu.VMEM((B,), jnp.int32),
        pltpu.VMEM((TFLAT,), jnp.float32),
    ],
    compiler_params=pltpu.CompilerParams(needs_layout_passes=False),
)
def _sc_stats(x_hbm, b_hbm, out_hbm, xb, idxb, table):
    cid = lax.axis_index("c")
    sid = lax.axis_index("s")
    half = sid % 2
    g8 = sid // 2
    zero16 = jnp.zeros((16,), jnp.float32)
    ones16 = jnp.full((16,), 1.0, jnp.float32)
    zero16i = jnp.zeros((16,), jnp.int32)
    lane = lax.iota(jnp.int32, 16)

    def zrow(k, c_):
        table[pl.ds(k * 16, 16)] = zero16
        return c_

    lax.fori_loop(0, TFLAT // 16, zrow, 0)

    def block_body(j, carry):
        b = cid + 2 * g8 + 16 * j

        @pl.when(b < NBT)
        def _():
            base = jnp.where(b == NB, TAIL_BASE, b * B)
            pltpu.sync_copy(b_hbm.at[pl.ds(base, B)], idxb)
            pltpu.sync_copy(x_hbm.at[pl.ds(base, B)], xb)
            r0 = jnp.where(b == NB, TAIL_SKIP, 0)

            @pl.when(half == 0)
            def _():
                def row_body(r, c_):
                    gvb = plsc.load_gather(idxb, [zero16i + r])
                    tb = gvb * TW + lane
                    for cc in range(CA // 16):
                        v = xb[r, pl.ds(cc * 16, 16)]
                        plsc.addupdate_scatter(table, [tb + cc * 16], v * v)
                        if cc < NSCAL // 16:
                            plsc.addupdate_scatter(
                                table, [tb + (CA + cc * 16)], v)
                    return c_

                lax.fori_loop(r0, B, row_body, 0)

            @pl.when(half == 1)
            def _():
                def row_body(r, c_):
                    gvb = plsc.load_gather(idxb, [zero16i + r])
                    tb = gvb * TW + lane
                    for cc in range(CB // 16):
                        v = xb[r, pl.ds(CA + cc * 16, 16)]
                        plsc.addupdate_scatter(table, [tb + cc * 16], v * v)
                    plsc.addupdate_scatter(table, [tb + CB], ones16)
                    return c_

                lax.fori_loop(r0, B, row_body, 0)

        return carry

    lax.fori_loop(0, MAXJ, block_body, 0)

    pltpu.sync_copy(table, out_hbm.at[pl.ds((cid * 16 + sid) * TFLAT, TFLAT)])


def _coeff_kernel(s_ref, m_ref, w_ref, bb_ref, coef_ref):
    st = s_ref[...]
    a = jnp.zeros((GP, TW), jnp.float32)
    bt = jnp.zeros((GP, TW), jnp.float32)
    for t in range(NW):
        part = st[t * GP:(t + 1) * GP]
        if t % 2 == 0:
            a = a + part
        else:
            bt = bt + part
    cnt = jnp.maximum(bt[:, CB:CB + 1], 1.0)
    mean = a[:, CA:CA + NSCAL] / cnt
    sq = jnp.concatenate([a[:, :CA], bt[:, :CB]], axis=1)
    gs = jax.lax.dot_general(
        sq, m_ref[...], (((1,), (0,)), ((), ())),
        preferred_element_type=jnp.float32, precision=_HI) / cnt
    meanp = jnp.concatenate(
        [mean, jnp.zeros((GP, D - NSCAL), jnp.float32)], axis=1)
    var = gs - meanp * meanp
    scale = w_ref[...] * jax.lax.rsqrt(var + EPS)
    shift = bb_ref[...] - mean * scale[:, :NSCAL]
    coef_ref[...] = jnp.concatenate(
        [scale, shift, jnp.zeros((GP, 16), jnp.float32)], axis=1)


@functools.partial(
    pl.kernel,
    mesh=_MESH,
    out_type=jax.ShapeDtypeStruct((N, D), jnp.float32),
    scratch_types=[
        pltpu.VMEM((B, D), jnp.float32),
        pltpu.VMEM((B, 256), jnp.float32),
        pltpu.VMEM((B,), jnp.int32),
        pltpu.SemaphoreType.DMA,
    ],
    compiler_params=pltpu.CompilerParams(needs_layout_passes=False),
)
def _sc_apply(x_hbm, b_hbm, coef_hbm, out_hbm, xb, cb, idxb, sem):
    cid = lax.axis_index("c")
    sid = lax.axis_index("s")
    wid = sid * 2 + cid

    def block_body(it, carry):
        b = wid + it * NW

        @pl.when(b < NBT)
        def _():
            base = jnp.where(b == NB, TAIL_BASE, b * B)
            pltpu.sync_copy(x_hbm.at[pl.ds(base, B)], xb)
            pltpu.sync_copy(b_hbm.at[pl.ds(base, B)], idxb)
            pltpu.async_copy(coef_hbm.at[idxb], cb, sem).wait()

            def row_body(r, rc):
                for c in range(D // 16):
                    v = xb[r, pl.ds(c * 16, 16)] * cb[r, pl.ds(c * 16, 16)]
                    if c < NSCAL // 16:
                        v = v + cb[r, pl.ds(D + c * 16, 16)]
                    xb[r, pl.ds(c * 16, 16)] = v
                return rc

            lax.fori_loop(0, B, row_body, 0)
            pltpu.sync_copy(xb, out_hbm.at[pl.ds(base, B)])

        return carry

    lax.fori_loop(0, MAXIT, block_body, 0)


@jax.jit
def kernel(input, batch, weight, bias):
    bi = batch.astype(jnp.int32)
    wcol = jnp.concatenate(
        [weight[:32], jnp.repeat(weight[32:64], 3), jnp.repeat(weight[64:], 5)]
    ).reshape(1, D)
    bb = bias.reshape(1, NSCAL)
    m = jnp.asarray(_M_CONST)

    stats = _sc_stats(input, bi)

    coef = pl.pallas_call(
        _coeff_kernel,
        out_shape=jax.ShapeDtypeStruct((GP, 256), jnp.float32),
    )(stats.reshape(NW * GP, TW), m, wcol, bb)

    return _sc_apply(input, bi, coef)


# speedup vs baseline: 2.7900x; 1.0820x over previous
import functools

import numpy as np
import jax
import jax.numpy as jnp
from jax import lax
from jax.experimental import pallas as pl
from jax.experimental.pallas import tpu as pltpu
from jax.experimental.pallas import tpu_sc as plsc

N = 100000
D = 208
G = 512
NSCAL = 32
EPS = 1e-5

B = 240
NB = N // B
TAIL_BASE = N - B
TAIL_SKIP = NB * B - TAIL_BASE
NBT = NB + 1
NW = 32
MAXIT = -(-NBT // NW)
GP = 512
TW = 128
CA = 96
CB = D - CA
MAXJ = -(-NBT // 16)
TAIL_RG = TAIL_SKIP // 16
TFLAT = GP * TW
MFLAT = 32 * TW

_HI = jax.lax.Precision.HIGHEST


def _build_group_mat():
    gid = np.zeros((D,), np.int32)
    dcol = np.zeros((D,), np.float32)
    c = 0
    f = 0
    for mul, l in ((32, 0), (32, 1), (16, 2)):
        d = 2 * l + 1
        for _ in range(mul):
            for _ in range(d):
                gid[c] = f
                dcol[c] = d
                c += 1
            f += 1
    return (gid[:, None] == gid[None, :]).astype(np.float32) / dcol[None, :]


_M_CONST = _build_group_mat()

_MESH = plsc.VectorSubcoreMesh(core_axis_name="c", subcore_axis_name="s")


@functools.partial(
    pl.kernel,
    mesh=_MESH,
    out_type=jax.ShapeDtypeStruct((NW * TFLAT,), jnp.float32),
    scratch_types=[
        pltpu.VMEM((B, D), jnp.float32),
        pltpu.VMEM((B,), jnp.int32),
        pltpu.VMEM((TFLAT,), jnp.float32),
    ],
    compiler_params=pltpu.CompilerParams(needs_layout_passes=False),
)
def _sc_stats(x_hbm, b_hbm, out_hbm, xb, idxb, table):
    cid = lax.axis_index("c")
    sid = lax.axis_index("s")
    half = sid % 2
    g8 = sid // 2
    zero16 = jnp.zeros((16,), jnp.float32)
    ones16 = jnp.full((16,), 1.0, jnp.float32)
    zero16i = jnp.zeros((16,), jnp.int32)
    lane = lax.iota(jnp.int32, 16)

    def zrow(k, c_):
        table[pl.ds(k * 16, 16)] = zero16
        return c_

    lax.fori_loop(0, TFLAT // 16, zrow, 0)

    def block_body(j, carry):
        b = cid + 2 * g8 + 16 * j

        @pl.when(b < NBT)
        def _():
            base = jnp.where(b == NB, TAIL_BASE, b * B)
            pltpu.sync_copy(b_hbm.at[pl.ds(base, B)], idxb)
            pltpu.sync_copy(x_hbm.at[pl.ds(base, B)], xb)
            r0 = jnp.where(b == NB, TAIL_SKIP, 0)

            @pl.when(half == 0)
            def _():
                def row_body(r, c_):
                    gvb = plsc.load_gather(idxb, [zero16i + r])
                    tb = gvb * TW + lane
                    for cc in range(CA // 16):
                        v = xb[r, pl.ds(cc * 16, 16)]
                        plsc.addupdate_scatter(table, [tb + cc * 16], v * v)
                        if cc < NSCAL // 16:
                            plsc.addupdate_scatter(
                                table, [tb + (CA + cc * 16)], v)
                    return c_

                lax.fori_loop(r0, B, row_body, 0)

            @pl.when(half == 1)
            def _():
                def row_body(r, c_):
                    gvb = plsc.load_gather(idxb, [zero16i + r])
                    tb = gvb * TW + lane
                    for cc in range(CB // 16):
                        v = xb[r, pl.ds(CA + cc * 16, 16)]
                        plsc.addupdate_scatter(table, [tb + cc * 16], v * v)
                    plsc.addupdate_scatter(table, [tb + CB], ones16)
                    return c_

                lax.fori_loop(r0, B, row_body, 0)

        return carry

    lax.fori_loop(0, MAXJ, block_body, 0)

    pltpu.sync_copy(table, out_hbm.at[pl.ds((cid * 16 + sid) * TFLAT, TFLAT)])


def _coeff_kernel(s_ref, m_ref, w_ref, bb_ref, coef_ref):
    st = s_ref[...]
    a = jnp.zeros((GP, TW), jnp.float32)
    bt = jnp.zeros((GP, TW), jnp.float32)
    for t in range(NW):
        part = st[t * GP:(t + 1) * GP]
        if t % 2 == 0:
            a = a + part
        else:
            bt = bt + part
    cnt = jnp.maximum(bt[:, CB:CB + 1], 1.0)
    mean = a[:, CA:CA + NSCAL] / cnt
    sq = jnp.concatenate([a[:, :CA], bt[:, :CB]], axis=1)
    gs = jax.lax.dot_general(
        sq, m_ref[...], (((1,), (0,)), ((), ())),
        preferred_element_type=jnp.float32, precision=_HI) / cnt
    meanp = jnp.concatenate(
        [mean, jnp.zeros((GP, D - NSCAL), jnp.float32)], axis=1)
    var = gs - meanp * meanp
    scale = w_ref[...] * jax.lax.rsqrt(var + EPS)
    shift = bb_ref[...] - mean * scale[:, :NSCAL]
    coef_ref[...] = jnp.concatenate(
        [scale, shift, jnp.zeros((GP, 16), jnp.float32)], axis=1)


@functools.partial(
    pl.kernel,
    mesh=_MESH,
    out_type=jax.ShapeDtypeStruct((N, D), jnp.float32),
    scratch_types=[
        pltpu.VMEM((B, D), jnp.float32),
        pltpu.VMEM((B, 256), jnp.float32),
        pltpu.VMEM((B,), jnp.int32),
        pltpu.SemaphoreType.DMA,
    ],
    compiler_params=pltpu.CompilerParams(needs_layout_passes=False),
)
def _sc_apply(x_hbm, b_hbm, coef_hbm, out_hbm, xb, cb, idxb, sem):
    cid = lax.axis_index("c")
    sid = lax.axis_index("s")
    wid = sid * 2 + cid

    def block_body(it, carry):
        b = wid + it * NW

        @pl.when(b < NBT)
        def _():
            base = jnp.where(b == NB, TAIL_BASE, b * B)
            pltpu.sync_copy(x_hbm.at[pl.ds(base, B)], xb)
            pltpu.sync_copy(b_hbm.at[pl.ds(base, B)], idxb)
            pltpu.async_copy(coef_hbm.at[idxb], cb, sem).wait()

            def row_body(r, rc):
                for c in range(D // 16):
                    v = xb[r, pl.ds(c * 16, 16)] * cb[r, pl.ds(c * 16, 16)]
                    if c < NSCAL // 16:
                        v = v + cb[r, pl.ds(D + c * 16, 16)]
                    xb[r, pl.ds(c * 16, 16)] = v
                return rc

            lax.fori_loop(0, B, row_body, 0)
            pltpu.sync_copy(xb, out_hbm.at[pl.ds(base, B)])

        return carry

    lax.fori_loop(0, MAXIT, block_body, 0)


@jax.jit
def kernel(input, batch, weight, bias):
    bi = batch.astype(jnp.int32)
    wcol = jnp.concatenate(
        [weight[:32], jnp.repeat(weight[32:64], 3), jnp.repeat(weight[64:], 5)]
    ).reshape(1, D)
    bb = bias.reshape(1, NSCAL)
    m = jnp.asarray(_M_CONST)

    stats = _sc_stats(input, bi)

    coef = pl.pallas_call(
        _coeff_kernel,
        out_shape=jax.ShapeDtypeStruct((GP, 256), jnp.float32),
    )(stats.reshape(NW * GP, TW), m, wcol, bb)

    return _sc_apply(input, bi, coef)


# speedup vs baseline: 3.3546x; 1.2023x over previous
import functools

import numpy as np
import jax
import jax.numpy as jnp
from jax import lax
from jax.experimental import pallas as pl
from jax.experimental.pallas import tpu as pltpu
from jax.experimental.pallas import tpu_sc as plsc

N = 100000
D = 208
G = 512
NSCAL = 32
EPS = 1e-5

B = 120
NB = N // B
TAIL_BASE = N - B
TAIL_SKIP = NB * B - TAIL_BASE
NBT = NB + 1
NW = 32
MAXIT = -(-NBT // NW)
GP = 512
TW = 128
CA = 96
CB = D - CA
MAXJ = -(-NBT // 16)
TAIL_RG = TAIL_SKIP // 16
TFLAT = GP * TW
MFLAT = 32 * TW

_HI = jax.lax.Precision.HIGHEST


def _build_group_mat():
    gid = np.zeros((D,), np.int32)
    dcol = np.zeros((D,), np.float32)
    c = 0
    f = 0
    for mul, l in ((32, 0), (32, 1), (16, 2)):
        d = 2 * l + 1
        for _ in range(mul):
            for _ in range(d):
                gid[c] = f
                dcol[c] = d
                c += 1
            f += 1
    return (gid[:, None] == gid[None, :]).astype(np.float32) / dcol[None, :]


_M_CONST = _build_group_mat()

_MESH = plsc.VectorSubcoreMesh(core_axis_name="c", subcore_axis_name="s")


@functools.partial(
    pl.kernel,
    mesh=_MESH,
    out_type=jax.ShapeDtypeStruct((NW * TFLAT,), jnp.float32),
    scratch_types=[
        pltpu.VMEM((B, D), jnp.float32),
        pltpu.VMEM((B, D), jnp.float32),
        pltpu.VMEM((B,), jnp.int32),
        pltpu.VMEM((B,), jnp.int32),
        pltpu.VMEM((TFLAT,), jnp.float32),
        pltpu.SemaphoreType.DMA,
        pltpu.SemaphoreType.DMA,
        pltpu.SemaphoreType.DMA,
        pltpu.SemaphoreType.DMA,
    ],
    compiler_params=pltpu.CompilerParams(needs_layout_passes=False),
)
def _sc_stats(x_hbm, b_hbm, out_hbm, xb0, xb1, ix0, ix1, table,
              sx0, sx1, si0, si1):
    cid = lax.axis_index("c")
    sid = lax.axis_index("s")
    half = sid % 2
    g8 = sid // 2
    zero16 = jnp.zeros((16,), jnp.float32)
    ones16 = jnp.full((16,), 1.0, jnp.float32)
    zero16i = jnp.zeros((16,), jnp.int32)
    lane = lax.iota(jnp.int32, 16)
    xbs, ixs, sxs, sis = [xb0, xb1], [ix0, ix1], [sx0, sx1], [si0, si1]

    def zrow(k, c_):
        table[pl.ds(k * 16, 16)] = zero16
        return c_

    lax.fori_loop(0, TFLAT // 16, zrow, 0)

    def bval(k):
        return cid + 2 * g8 + 16 * k

    def bbase(k):
        return jnp.where(bval(k) == NB, TAIL_BASE, bval(k) * B)

    def issue_load(k, p):
        @pl.when(bval(k) < NBT)
        def _():
            pltpu.async_copy(x_hbm.at[pl.ds(bbase(k), B)], xbs[p], sxs[p])
            pltpu.async_copy(b_hbm.at[pl.ds(bbase(k), B)], ixs[p], sis[p])

    def step(k, p):
        xb, idxb = xbs[p], ixs[p]

        @pl.when(bval(k) < NBT)
        def _():
            pltpu.make_async_copy(
                x_hbm.at[pl.ds(bbase(k), B)], xb, sxs[p]).wait()
            pltpu.make_async_copy(
                b_hbm.at[pl.ds(bbase(k), B)], idxb, sis[p]).wait()
            r0 = jnp.where(bval(k) == NB, TAIL_SKIP, 0)

            @pl.when(half == 0)
            def _():
                def row_body(r, c_):
                    gvb = plsc.load_gather(idxb, [zero16i + r])
                    tb = gvb * TW + lane
                    for cc in range(CA // 16):
                        v = xb[r, pl.ds(cc * 16, 16)]
                        plsc.addupdate_scatter(table, [tb + cc * 16], v * v)
                        if cc < NSCAL // 16:
                            plsc.addupdate_scatter(
                                table, [tb + (CA + cc * 16)], v)
                    return c_

                lax.fori_loop(r0, B, row_body, 0)

            @pl.when(half == 1)
            def _():
                def row_body(r, c_):
                    gvb = plsc.load_gather(idxb, [zero16i + r])
                    tb = gvb * TW + lane
                    for cc in range(CB // 16):
                        v = xb[r, pl.ds(CA + cc * 16, 16)]
                        plsc.addupdate_scatter(table, [tb + cc * 16], v * v)
                    plsc.addupdate_scatter(table, [tb + CB], ones16)
                    return c_

                lax.fori_loop(r0, B, row_body, 0)

        issue_load(k + 2, p)

    issue_load(0, 0)
    issue_load(1, 1)

    def pair_body(kk, c_):
        step(2 * kk, 0)
        step(2 * kk + 1, 1)
        return c_

    lax.fori_loop(0, -(-MAXJ // 2), pair_body, 0)

    pltpu.sync_copy(table, out_hbm.at[pl.ds((cid * 16 + sid) * TFLAT, TFLAT)])


def _coeff_kernel(s_ref, m_ref, w_ref, bb_ref, coef_ref):
    st = s_ref[...]
    a = jnp.zeros((GP, TW), jnp.float32)
    bt = jnp.zeros((GP, TW), jnp.float32)
    for t in range(NW):
        part = st[t * GP:(t + 1) * GP]
        if t % 2 == 0:
            a = a + part
        else:
            bt = bt + part
    cnt = jnp.maximum(bt[:, CB:CB + 1], 1.0)
    mean = a[:, CA:CA + NSCAL] / cnt
    sq = jnp.concatenate([a[:, :CA], bt[:, :CB]], axis=1)
    gs = jax.lax.dot_general(
        sq, m_ref[...], (((1,), (0,)), ((), ())),
        preferred_element_type=jnp.float32, precision=_HI) / cnt
    meanp = jnp.concatenate(
        [mean, jnp.zeros((GP, D - NSCAL), jnp.float32)], axis=1)
    var = gs - meanp * meanp
    scale = w_ref[...] * jax.lax.rsqrt(var + EPS)
    shift = bb_ref[...] - mean * scale[:, :NSCAL]
    coef_ref[...] = jnp.concatenate(
        [scale, shift, jnp.zeros((GP, 16), jnp.float32)], axis=1)


@functools.partial(
    pl.kernel,
    mesh=_MESH,
    out_type=jax.ShapeDtypeStruct((N, D), jnp.float32),
    scratch_types=[
        pltpu.VMEM((B, D), jnp.float32),
        pltpu.VMEM((B, D), jnp.float32),
        pltpu.VMEM((B, 256), jnp.float32),
        pltpu.VMEM((B, D), jnp.float32),
        pltpu.VMEM((B,), jnp.int32),
        pltpu.VMEM((B,), jnp.int32),
        pltpu.SemaphoreType.DMA,
        pltpu.SemaphoreType.DMA,
        pltpu.SemaphoreType.DMA,
        pltpu.SemaphoreType.DMA,
        pltpu.SemaphoreType.DMA,
        pltpu.SemaphoreType.DMA,
    ],
    compiler_params=pltpu.CompilerParams(needs_layout_passes=False),
)
def _sc_apply(x_hbm, b_hbm, coef_hbm, out_hbm, xb0, xb1, cb, ob, ix0, ix1,
              sx0, sx1, si0, si1, sc, so):
    cid = lax.axis_index("c")
    sid = lax.axis_index("s")
    wid = sid * 2 + cid
    xbs, ixs, sxs, sis = [xb0, xb1], [ix0, ix1], [sx0, sx1], [si0, si1]

    def bval(k):
        return wid + k * NW

    def bbase(k):
        return jnp.where(bval(k) == NB, TAIL_BASE, bval(k) * B)

    def issue_load(k, p):
        @pl.when(bval(k) < NBT)
        def _():
            pltpu.async_copy(x_hbm.at[pl.ds(bbase(k), B)], xbs[p], sxs[p])
            pltpu.async_copy(b_hbm.at[pl.ds(bbase(k), B)], ixs[p], sis[p])

    def wait_idx_issue_gather(k, p):
        @pl.when(bval(k) < NBT)
        def _():
            pltpu.make_async_copy(
                b_hbm.at[pl.ds(bbase(k), B)], ixs[p], sis[p]).wait()
            pltpu.async_copy(coef_hbm.at[ixs[p]], cb, sc)

    def wait_out(k):
        pltpu.make_async_copy(ob, out_hbm.at[pl.ds(bbase(k), B)], so).wait()

    def step(k, p):
        xb = xbs[p]

        @pl.when(bval(k) < NBT)
        def _():
            pltpu.make_async_copy(
                x_hbm.at[pl.ds(bbase(k), B)], xb, sxs[p]).wait()
            pltpu.make_async_copy(coef_hbm.at[ixs[p]], cb, sc).wait()

            @pl.when(k >= 1)
            def _():
                wait_out(k - 1)

            def row_body(r, rc):
                for c in range(D // 16):
                    v = xb[r, pl.ds(c * 16, 16)] * cb[r, pl.ds(c * 16, 16)]
                    if c < NSCAL // 16:
                        v = v + cb[r, pl.ds(D + c * 16, 16)]
                    ob[r, pl.ds(c * 16, 16)] = v
                return rc

            lax.fori_loop(0, B, row_body, 0)
            pltpu.async_copy(ob, out_hbm.at[pl.ds(bbase(k), B)], so)

        issue_load(k + 2, p)
        wait_idx_issue_gather(k + 1, 1 - p)

    issue_load(0, 0)
    issue_load(1, 1)
    wait_idx_issue_gather(0, 0)

    def pair_body(kk, c_):
        step(2 * kk, 0)
        step(2 * kk + 1, 1)
        return c_

    lax.fori_loop(0, -(-MAXIT // 2), pair_body, 0)
    wait_out((NBT - 1 - wid) // NW)


@jax.jit
def kernel(input, batch, weight, bias):
    bi = batch.astype(jnp.int32)
    wcol = jnp.concatenate(
        [weight[:32], jnp.repeat(weight[32:64], 3), jnp.repeat(weight[64:], 5)]
    ).reshape(1, D)
    bb = bias.reshape(1, NSCAL)
    m = jnp.asarray(_M_CONST)

    stats = _sc_stats(input, bi)

    coef = pl.pallas_call(
        _coeff_kernel,
        out_shape=jax.ShapeDtypeStruct((GP, 256), jnp.float32),
    )(stats.reshape(NW * GP, TW), m, wcol, bb)

    return _sc_apply(input, bi, coef)
